# baseline (speedup 1.0000x reference)
import functools

import jax
import jax.numpy as jnp
from jax import lax
from jax.experimental import pallas as pl
from jax.experimental.pallas import tpu as pltpu
from jax.experimental.pallas import tpu_sc as plsc

F32 = jnp.float32
I32 = jnp.int32

B, L, A = 1024, 50, 20
GPS_D, USER_D, WD_D, AOI_D, ATY_D, DIPAN_D, CONTI_D = 32, 64, 8, 64, 16, 16, 16
N_ATY, N_DIPAN = 21, 1001

NC, NS = 2, 16
NW = NC * NS
CHUNK = 128
NBUF = 6

N_UNPICK = B * L
N_AOI = B * A
N_BIG = N_UNPICK + N_AOI + B
N_PAD = NW * CHUNK * ((N_BIG + NW * CHUNK - 1) // (NW * CHUNK))
CPW = N_PAD // (NW * CHUNK)


def _sc_body(aoi_table, user_table, aoi_idx, user_idx,
             aoi_rows, user_rows,
             idx_v, uidx_v, u_buf, bufs, gsems, wsems, usem):
    wid = lax.axis_index("s") * NC + lax.axis_index("c")
    base_c = wid * CPW
    pltpu.sync_copy(aoi_idx.at[wid], idx_v)

    wcopies = [None] * NBUF
    gcopies = [None] * NBUF
    for w in range(CPW // NBUF):
        for s in range(NBUF):
            j = w * NBUF + s
            if w > 0:
                wcopies[s].wait()
            gcopies[s] = pltpu.async_copy(
                aoi_table.at[idx_v.at[j]], bufs.at[s], gsems.at[s])
        for s in range(NBUF):
            j = w * NBUF + s
            gcopies[s].wait()
            row0 = (base_c + j) * CHUNK
            wcopies[s] = pltpu.async_copy(
                bufs.at[s], aoi_rows.at[pl.ds(row0, CHUNK)], wsems.at[s])
    for s in range(NBUF):
        wcopies[s].wait()

    n_sm = B // CHUNK

    @pl.when(wid < n_sm)
    def _():
        pltpu.sync_copy(user_idx.at[wid], uidx_v)
        pltpu.async_copy(user_table.at[uidx_v.at[0]], u_buf, usem).wait()
        pltpu.sync_copy(u_buf, user_rows.at[pl.ds(wid * CHUNK, CHUNK)])


@functools.cache
def _make_sc_gather():
    return pl.kernel(
        _sc_body,
        out_type=(
            jax.ShapeDtypeStruct((N_PAD, AOI_D), F32),
            jax.ShapeDtypeStruct((B, USER_D), F32),
        ),
        mesh=plsc.VectorSubcoreMesh(core_axis_name="c", subcore_axis_name="s",
                                    num_cores=NC, num_subcores=NS),
        compiler_params=pltpu.CompilerParams(use_tc_tiling_on_sc=False),
        scratch_types=[
            pltpu.VMEM((CPW, CHUNK), I32),
            pltpu.VMEM((1, CHUNK), I32),
            pltpu.VMEM((CHUNK, USER_D), F32),
            pltpu.VMEM((NBUF, CHUNK, AOI_D), F32),
            pltpu.SemaphoreType.DMA((NBUF,)),
            pltpu.SemaphoreType.DMA((NBUF,)),
            pltpu.SemaphoreType.DMA,
        ],
    )


def _dot(x, w):
    return lax.dot_general(x, w, (((1,), (0,)), ((), ())),
                           precision=lax.Precision.HIGHEST,
                           preferred_element_type=F32)


def _onehot_embed(idx_f32, table, n_rows):
    idx = idx_f32.astype(I32)
    rows = idx.shape[0]
    oh = (lax.broadcasted_iota(I32, (rows, n_rows), 1) == idx[:, None])
    return _dot(oh.astype(F32), table)


def _unpick_body(fea, aoi, wg, bg, wu, bu, aty_tab, out):
    gps = _dot(fea[:, 0:2], wg[...]) + bg[...]
    conti = _dot(fea[:, 4:10], wu[...]) + bu[...]
    aty = _onehot_embed(fea[:, 3], aty_tab[...], N_ATY)
    res = jnp.concatenate([gps, aoi[...], aty, conti], axis=1)
    bb = out.shape[0]
    for b in range(bb):
        out[b] = res[L * b:L * (b + 1), :]


def _aoi_body(fea, aoi, wg, bg, wa, ba, aty_tab, out):
    gps = _dot(fea[:, 2:4], wg[...]) + bg[...]
    conti = _dot(fea[:, 4:12], wa[...]) + ba[...]
    aty = _onehot_embed(fea[:, 1], aty_tab[...], N_ATY)
    res = jnp.concatenate([gps, aoi[...], aty, conti], axis=1)
    bb = out.shape[0]
    for b in range(bb):
        out[b] = res[A * b:A * (b + 1), :]


def _glob_body(g, aoi, user, wg, bg, wc, bc, wd_tab, aty_tab, dipan_tab,
               courier, glob):
    gps1 = _dot(g[:, 5:7], wg[...]) + bg[...]
    gps2 = _dot(g[:, 9:11], wg[...]) + bg[...]
    aty = _onehot_embed(g[:, 8], aty_tab[...], N_ATY)
    courier[...] = jnp.concatenate([gps1, gps2, aoi[...], aty], axis=1)
    conti = _dot(jnp.concatenate([g[:, 1:3], g[:, 4:5]], axis=1), wc[...]) + bc[...]
    wd_emb = _onehot_embed(g[:, 3], wd_tab[...], 8)
    dipan = _onehot_embed(g[:, 11], dipan_tab[...], N_DIPAN)
    glob[...] = jnp.concatenate([conti, user[...], wd_emb, dipan], axis=1)


def kernel(unpick_fea, edge_fea, unpick_len, last_fea, last_len, global_fea,
           idx, pos, aoi_index, aoi_fea, aoi_edge, aoi_len, aoi_idx, aoi_pos,
           W_gps, b_gps, user_table, weekday_table, aoi_table, aoi_type_table,
           dipan_table, W_gconti, b_gconti, W_uconti, b_uconti, W_aconti,
           b_aconti):
    pad = jnp.zeros((N_PAD - N_BIG,), I32)
    big_aoi_idx = jnp.concatenate([
        unpick_fea[:, :, 2].astype(I32).reshape(-1),
        aoi_fea[:, :, 0].astype(I32).reshape(-1),
        global_fea[:, 7].astype(I32),
        pad,
    ]).reshape(NW, CPW, CHUNK)
    user_idx = global_fea[:, 0].astype(I32).reshape(B // CHUNK, 1, CHUNK)

    aoi_rows, user_rows = _make_sc_gather()(
        aoi_table, user_table, big_aoi_idx, user_idx)

    b_gps2 = b_gps.reshape(1, GPS_D)
    out_d = GPS_D + AOI_D + ATY_D + CONTI_D

    bb = 128
    unpick_new = pl.pallas_call(
        _unpick_body,
        grid=(B // bb,),
        in_specs=[
            pl.BlockSpec((bb * L, 10), lambda i: (i, 0)),
            pl.BlockSpec((bb * L, AOI_D), lambda i: (i, 0)),
            pl.BlockSpec((2, GPS_D), lambda i: (0, 0)),
            pl.BlockSpec((1, GPS_D), lambda i: (0, 0)),
            pl.BlockSpec((6, CONTI_D), lambda i: (0, 0)),
            pl.BlockSpec((1, CONTI_D), lambda i: (0, 0)),
            pl.BlockSpec((N_ATY, ATY_D), lambda i: (0, 0)),
        ],
        out_specs=pl.BlockSpec((bb, L, out_d), lambda i: (i, 0, 0)),
        out_shape=jax.ShapeDtypeStruct((B, L, out_d), F32),
    )(unpick_fea.reshape(N_UNPICK, 10), aoi_rows,
      W_gps, b_gps2, W_uconti, b_uconti.reshape(1, CONTI_D), aoi_type_table)

    aoi_blk_off = N_UNPICK // (bb * A)
    aoi_new = pl.pallas_call(
        _aoi_body,
        grid=(B // bb,),
        in_specs=[
            pl.BlockSpec((bb * A, 12), lambda i: (i, 0)),
            pl.BlockSpec((bb * A, AOI_D), lambda i: (i + aoi_blk_off, 0)),
            pl.BlockSpec((2, GPS_D), lambda i: (0, 0)),
            pl.BlockSpec((1, GPS_D), lambda i: (0, 0)),
            pl.BlockSpec((8, CONTI_D), lambda i: (0, 0)),
            pl.BlockSpec((1, CONTI_D), lambda i: (0, 0)),
            pl.BlockSpec((N_ATY, ATY_D), lambda i: (0, 0)),
        ],
        out_specs=pl.BlockSpec((bb, A, out_d), lambda i: (i, 0, 0)),
        out_shape=jax.ShapeDtypeStruct((B, A, out_d), F32),
    )(aoi_fea.reshape(N_AOI, 12), aoi_rows,
      W_gps, b_gps2, W_aconti, b_aconti.reshape(1, CONTI_D), aoi_type_table)

    glb_off = (N_UNPICK + N_AOI) // B
    courier, glob = pl.pallas_call(
        _glob_body,
        grid=(1,),
        in_specs=[
            pl.BlockSpec((B, 12), lambda i: (0, 0)),
            pl.BlockSpec((B, AOI_D), lambda i: (glb_off, 0)),
            pl.BlockSpec((B, USER_D), lambda i: (0, 0)),
            pl.BlockSpec((2, GPS_D), lambda i: (0, 0)),
            pl.BlockSpec((1, GPS_D), lambda i: (0, 0)),
            pl.BlockSpec((3, CONTI_D), lambda i: (0, 0)),
            pl.BlockSpec((1, CONTI_D), lambda i: (0, 0)),
            pl.BlockSpec((8, WD_D), lambda i: (0, 0)),
            pl.BlockSpec((N_ATY, ATY_D), lambda i: (0, 0)),
            pl.BlockSpec((N_DIPAN, DIPAN_D), lambda i: (0, 0)),
        ],
        out_specs=[
            pl.BlockSpec((B, 2 * GPS_D + AOI_D + ATY_D), lambda i: (0, 0)),
            pl.BlockSpec((B, CONTI_D + USER_D + WD_D + DIPAN_D), lambda i: (0, 0)),
        ],
        out_shape=[
            jax.ShapeDtypeStruct((B, 2 * GPS_D + AOI_D + ATY_D), F32),
            jax.ShapeDtypeStruct((B, CONTI_D + USER_D + WD_D + DIPAN_D), F32),
        ],
    )(global_fea, aoi_rows, user_rows,
      W_gps, b_gps2, W_gconti, b_gconti.reshape(1, CONTI_D), weekday_table,
      aoi_type_table, dipan_table)

    return unpick_new, aoi_new, courier, glob


# speedup vs baseline: 1.0056x; 1.0056x over previous
import functools

import jax
import jax.numpy as jnp
from jax import lax
from jax.experimental import pallas as pl
from jax.experimental.pallas import tpu as pltpu
from jax.experimental.pallas import tpu_sc as plsc

F32 = jnp.float32
I32 = jnp.int32

B, L, A = 1024, 50, 20
GPS_D, USER_D, WD_D, AOI_D, ATY_D, DIPAN_D, CONTI_D = 32, 64, 8, 64, 16, 16, 16
N_ATY, N_DIPAN = 21, 1001

NC, NS = 2, 16
NW = NC * NS
CHUNK = 128
NBUF = 6

N_UNPICK = B * L
N_AOI = B * A
N_BIG = N_UNPICK + N_AOI + B
N_PAD = NW * CHUNK * ((N_BIG + NW * CHUNK - 1) // (NW * CHUNK))
CPW = N_PAD // (NW * CHUNK)


def _sc_body(aoi_table, user_table, aoi_idx, user_idx,
             aoi_rows, user_rows,
             idx_v, uidx_v, u_buf, bufs, gsems, wsems, usem):
    wid = lax.axis_index("s") * NC + lax.axis_index("c")
    base_c = wid * CPW
    pltpu.sync_copy(aoi_idx.at[wid], idx_v)

    wcopies = [None] * NBUF
    gcopies = [None] * NBUF
    for w in range(CPW // NBUF):
        for s in range(NBUF):
            j = w * NBUF + s
            if w > 0:
                wcopies[s].wait()
            gcopies[s] = pltpu.async_copy(
                aoi_table.at[idx_v.at[j]], bufs.at[s], gsems.at[s])
        for s in range(NBUF):
            j = w * NBUF + s
            gcopies[s].wait()
            row0 = (base_c + j) * CHUNK
            wcopies[s] = pltpu.async_copy(
                bufs.at[s], aoi_rows.at[pl.ds(row0, CHUNK)], wsems.at[s])
    for s in range(NBUF):
        wcopies[s].wait()

    n_sm = B // CHUNK

    @pl.when(wid < n_sm)
    def _():
        pltpu.sync_copy(user_idx.at[wid], uidx_v)
        pltpu.async_copy(user_table.at[uidx_v.at[0]], u_buf, usem).wait()
        pltpu.sync_copy(u_buf, user_rows.at[pl.ds(wid * CHUNK, CHUNK)])


@functools.cache
def _make_sc_gather():
    return pl.kernel(
        _sc_body,
        out_type=(
            jax.ShapeDtypeStruct((N_PAD, AOI_D), F32),
            jax.ShapeDtypeStruct((B, USER_D), F32),
        ),
        mesh=plsc.VectorSubcoreMesh(core_axis_name="c", subcore_axis_name="s",
                                    num_cores=NC, num_subcores=NS),
        compiler_params=pltpu.CompilerParams(use_tc_tiling_on_sc=False),
        scratch_types=[
            pltpu.VMEM((CPW, CHUNK), I32),
            pltpu.VMEM((1, CHUNK), I32),
            pltpu.VMEM((CHUNK, USER_D), F32),
            pltpu.VMEM((NBUF, CHUNK, AOI_D), F32),
            pltpu.SemaphoreType.DMA((NBUF,)),
            pltpu.SemaphoreType.DMA((NBUF,)),
            pltpu.SemaphoreType.DMA,
        ],
    )


def _dot(x, w):
    return lax.dot_general(x, w, (((1,), (0,)), ((), ())),
                           precision=lax.Precision.HIGHEST,
                           preferred_element_type=F32)


def _onehot_embed(idx_f32, table, n_rows):
    idx = idx_f32.astype(I32)
    rows = idx.shape[0]
    oh = (lax.broadcasted_iota(I32, (rows, n_rows), 1) == idx[:, None])
    return _dot(oh.astype(F32), table)


def _unpick_body(fea, aoi, wg, bg, wu, bu, aty_tab, out):
    gps = _dot(fea[:, 0:2], wg[...]) + bg[...]
    conti = _dot(fea[:, 4:10], wu[...]) + bu[...]
    aty = _onehot_embed(fea[:, 3], aty_tab[...], N_ATY)
    res = jnp.concatenate([gps, aoi[...], aty, conti], axis=1)
    bb = out.shape[0]
    for b in range(bb):
        out[b] = res[L * b:L * (b + 1), :]


def _aoi_body(fea, aoi, wg, bg, wa, ba, aty_tab, out):
    gps = _dot(fea[:, 2:4], wg[...]) + bg[...]
    conti = _dot(fea[:, 4:12], wa[...]) + ba[...]
    aty = _onehot_embed(fea[:, 1], aty_tab[...], N_ATY)
    res = jnp.concatenate([gps, aoi[...], aty, conti], axis=1)
    bb = out.shape[0]
    for b in range(bb):
        out[b] = res[A * b:A * (b + 1), :]


def _glob_body(g, aoi, user, wg, bg, wc, bc, wd_tab, aty_tab, dipan_tab,
               courier, glob):
    gps1 = _dot(g[:, 5:7], wg[...]) + bg[...]
    gps2 = _dot(g[:, 9:11], wg[...]) + bg[...]
    aty = _onehot_embed(g[:, 8], aty_tab[...], N_ATY)
    courier[...] = jnp.concatenate([gps1, gps2, aoi[...], aty], axis=1)
    conti = _dot(jnp.concatenate([g[:, 1:3], g[:, 4:5]], axis=1), wc[...]) + bc[...]
    wd_emb = _onehot_embed(g[:, 3], wd_tab[...], 8)
    dipan = _onehot_embed(g[:, 11], dipan_tab[...], N_DIPAN)
    glob[...] = jnp.concatenate([conti, user[...], wd_emb, dipan], axis=1)


def kernel(unpick_fea, edge_fea, unpick_len, last_fea, last_len, global_fea,
           idx, pos, aoi_index, aoi_fea, aoi_edge, aoi_len, aoi_idx, aoi_pos,
           W_gps, b_gps, user_table, weekday_table, aoi_table, aoi_type_table,
           dipan_table, W_gconti, b_gconti, W_uconti, b_uconti, W_aconti,
           b_aconti):
    pad = jnp.zeros((N_PAD - N_BIG,), I32)
    big_aoi_idx = jnp.concatenate([
        unpick_fea[:, :, 2].astype(I32).reshape(-1),
        aoi_fea[:, :, 0].astype(I32).reshape(-1),
        global_fea[:, 7].astype(I32),
        pad,
    ]).reshape(NW, CPW, CHUNK)
    user_idx = global_fea[:, 0].astype(I32).reshape(B // CHUNK, 1, CHUNK)

    aoi_rows, user_rows = _make_sc_gather()(
        aoi_table, user_table, big_aoi_idx, user_idx)

    b_gps2 = b_gps.reshape(1, GPS_D)
    out_d = GPS_D + AOI_D + ATY_D + CONTI_D

    bb = 64
    unpick_new = pl.pallas_call(
        _unpick_body,
        grid=(B // bb,),
        in_specs=[
            pl.BlockSpec((bb * L, 10), lambda i: (i, 0)),
            pl.BlockSpec((bb * L, AOI_D), lambda i: (i, 0)),
            pl.BlockSpec((2, GPS_D), lambda i: (0, 0)),
            pl.BlockSpec((1, GPS_D), lambda i: (0, 0)),
            pl.BlockSpec((6, CONTI_D), lambda i: (0, 0)),
            pl.BlockSpec((1, CONTI_D), lambda i: (0, 0)),
            pl.BlockSpec((N_ATY, ATY_D), lambda i: (0, 0)),
        ],
        out_specs=pl.BlockSpec((bb, L, out_d), lambda i: (i, 0, 0)),
        out_shape=jax.ShapeDtypeStruct((B, L, out_d), F32),
    )(unpick_fea.reshape(N_UNPICK, 10), aoi_rows,
      W_gps, b_gps2, W_uconti, b_uconti.reshape(1, CONTI_D), aoi_type_table)

    aoi_blk_off = N_UNPICK // (bb * A)
    aoi_new = pl.pallas_call(
        _aoi_body,
        grid=(B // bb,),
        in_specs=[
            pl.BlockSpec((bb * A, 12), lambda i: (i, 0)),
            pl.BlockSpec((bb * A, AOI_D), lambda i: (i + aoi_blk_off, 0)),
            pl.BlockSpec((2, GPS_D), lambda i: (0, 0)),
            pl.BlockSpec((1, GPS_D), lambda i: (0, 0)),
            pl.BlockSpec((8, CONTI_D), lambda i: (0, 0)),
            pl.BlockSpec((1, CONTI_D), lambda i: (0, 0)),
            pl.BlockSpec((N_ATY, ATY_D), lambda i: (0, 0)),
        ],
        out_specs=pl.BlockSpec((bb, A, out_d), lambda i: (i, 0, 0)),
        out_shape=jax.ShapeDtypeStruct((B, A, out_d), F32),
    )(aoi_fea.reshape(N_AOI, 12), aoi_rows,
      W_gps, b_gps2, W_aconti, b_aconti.reshape(1, CONTI_D), aoi_type_table)

    glb_off = (N_UNPICK + N_AOI) // B
    courier, glob = pl.pallas_call(
        _glob_body,
        grid=(1,),
        in_specs=[
            pl.BlockSpec((B, 12), lambda i: (0, 0)),
            pl.BlockSpec((B, AOI_D), lambda i: (glb_off, 0)),
            pl.BlockSpec((B, USER_D), lambda i: (0, 0)),
            pl.BlockSpec((2, GPS_D), lambda i: (0, 0)),
            pl.BlockSpec((1, GPS_D), lambda i: (0, 0)),
            pl.BlockSpec((3, CONTI_D), lambda i: (0, 0)),
            pl.BlockSpec((1, CONTI_D), lambda i: (0, 0)),
            pl.BlockSpec((8, WD_D), lambda i: (0, 0)),
            pl.BlockSpec((N_ATY, ATY_D), lambda i: (0, 0)),
            pl.BlockSpec((N_DIPAN, DIPAN_D), lambda i: (0, 0)),
        ],
        out_specs=[
            pl.BlockSpec((B, 2 * GPS_D + AOI_D + ATY_D), lambda i: (0, 0)),
            pl.BlockSpec((B, CONTI_D + USER_D + WD_D + DIPAN_D), lambda i: (0, 0)),
        ],
        out_shape=[
            jax.ShapeDtypeStruct((B, 2 * GPS_D + AOI_D + ATY_D), F32),
            jax.ShapeDtypeStruct((B, CONTI_D + USER_D + WD_D + DIPAN_D), F32),
        ],
    )(global_fea, aoi_rows, user_rows,
      W_gps, b_gps2, W_gconti, b_gconti.reshape(1, CONTI_D), weekday_table,
      aoi_type_table, dipan_table)

    return unpick_new, aoi_new, courier, glob
